# baseline (speedup 1.0000x reference)
import functools

import jax
import jax.numpy as jnp
import numpy as np
from jax.experimental import pallas as pl
from jax.experimental.pallas import tpu as pltpu


def _match_kernel(targets_ref, tdat_ref, priors_ref, ov_ref, clsa_ref,
                  enc_ref, *, nobj, num_p, num_c):
    t = targets_ref[0]
    pr = priors_ref[...]

    tx1 = t[:, 0:1]
    ty1 = t[:, 1:2]
    tx2 = t[:, 2:3]
    ty2 = t[:, 3:4]

    pcx = pr[0:1, :]
    pcy = pr[1:2, :]
    pw = pr[2:3, :]
    ph = pr[3:4, :]
    px1 = pcx - pw * 0.5
    py1 = pcy - ph * 0.5
    px2 = pcx + pw * 0.5
    py2 = pcy + ph * 0.5

    iw = jnp.maximum(jnp.minimum(tx2, px2) - jnp.maximum(tx1, px1), 0.0)
    ih = jnp.maximum(jnp.minimum(ty2, py2) - jnp.maximum(ty1, py1), 0.0)
    inter = iw * ih
    area_t = (tx2 - tx1) * (ty2 - ty1)
    area_p = (px2 - px1) * (py2 - py1)
    union = jnp.maximum(area_t + area_p - inter, 1e-8)
    ov = inter / union

    iota_t = jax.lax.broadcasted_iota(jnp.int32, (nobj, num_p), 0)
    iota_p = jax.lax.broadcasted_iota(jnp.int32, (nobj, num_p), 1)

    btv = jnp.max(ov, axis=0, keepdims=True)
    bpv = jnp.max(ov, axis=1, keepdims=True)
    bpi = jnp.min(jnp.where(ov == bpv, iota_p, num_p), axis=1,
                  keepdims=True)

    oh2 = jnp.where(iota_p == bpi, 1.0, 0.0)
    rr = jax.lax.broadcasted_iota(jnp.int32, (2, nobj), 0)
    cc = jax.lax.broadcasted_iota(jnp.int32, (2, nobj), 1)
    e = cc - 16 + 16 * rr
    valid = jnp.logical_and(e >= 0, e < 16)
    pow2e = jax.lax.bitcast_convert_type(
        jax.lax.shift_left(e + 127, 23), jnp.float32)
    wmat = jnp.where(valid, pow2e, 0.0)
    s2 = jnp.dot(wmat, oh2, preferred_element_type=jnp.float32)
    s_hi = s2[0:1, :]
    s_lo = s2[1:2, :]
    forced = (s_hi + s_lo) > 0.0
    e_hi = jax.lax.shift_right_logical(
        jax.lax.bitcast_convert_type(s_hi, jnp.int32), 23) - (127 - 16)
    e_lo = jax.lax.shift_right_logical(
        jax.lax.bitcast_convert_type(s_lo, jnp.int32), 23) - 127
    tf_i = jnp.where(s_hi > 0.0, e_hi, e_lo)
    tf_i = jnp.where(forced, tf_i, -1)
    fov = jnp.where(forced, 2.0, btv)

    f_b = jnp.where(forced, 1.0, 0.0)
    mi_f = jnp.where(iota_t == tf_i, 1.0, 0.0)
    mv_f = jnp.where(ov == btv, 1.0, 0.0)
    oh = mv_f + f_b * (mi_f - mv_f)
    gathered = jnp.dot(tdat_ref[0], oh,
                       preferred_element_type=jnp.float32)
    g_x1 = gathered[0:1, :]
    g_y1 = gathered[1:2, :]
    g_x2 = gathered[2:3, :]
    g_y2 = gathered[3:4, :]
    g_lab = gathered[4:5, :]

    inv_pw = 1.0 / pw
    inv_ph = 1.0 / ph
    e_cx = ((g_x1 + g_x2) * 0.5 - pcx) * (10.0 * inv_pw)
    e_cy = ((g_y1 + g_y2) * 0.5 - pcy) * (10.0 * inv_ph)
    e_w = jnp.log(jnp.maximum((g_x2 - g_x1) * inv_pw, 1e-8)) * 5.0
    e_h = jnp.log(jnp.maximum((g_y2 - g_y1) * inv_ph, 1e-8)) * 5.0

    cls_i = (g_lab + 0.5).astype(jnp.int32)
    act = fov >= 0.4
    neg1 = jnp.full((1, num_p), -1, jnp.int32)

    ov_ref[0] = fov
    clsa_ref[0] = jnp.where(act, cls_i, neg1)
    enc_ref[0, 0:1, :] = e_cx
    enc_ref[0, 1:2, :] = e_cy
    enc_ref[0, 2:3, :] = e_w
    enc_ref[0, 3:4, :] = e_h


_B_L1 = float(np.e ** (1.5 / 0.5) - 1.0)


_LN2 = float(np.log(2.0))


def _loss_kernel(conf_ref, confn_ref, loc_ref, enc_ref, ovr_ref, clsa_ref,
                 sl_ref, sc_ref, sp_ref, xsel_ref, *, pblk, num_c):
    b = pl.program_id(0)
    j = pl.program_id(1)

    @pl.when(jnp.logical_and(b == 0, j == 0))
    def _init():
        zero11 = jnp.zeros((1, 1), jnp.float32)
        sl_ref[...] = zero11
        sc_ref[...] = zero11
        sp_ref[...] = zero11

    x = conf_ref[0, 0]

    u = jnp.exp(-x)
    d = 1.0 + u
    lg = jnp.log2(d)
    el = lg * _LN2
    r2 = jnp.exp2(lg * -2.0)
    f_neg = (x + el) * r2
    sc_ref[...] = sc_ref[...] + jnp.sum(f_neg)

    xn = confn_ref[0]
    iota_c = jax.lax.broadcasted_iota(jnp.int32, (pblk, num_c), 1)
    selm = iota_c == clsa_ref[0]
    xsel_ref[0] = jnp.sum(jnp.where(selm, xn, 0.0), axis=1, keepdims=True)

    ov_row = ovr_ref[0]
    pos_row = ov_row >= 0.5
    sp_ref[...] = sp_ref[...] + jnp.sum(jnp.where(pos_row, 1.0, 0.0))

    pred = loc_ref[0]
    tgt = enc_ref[0]
    diff = jnp.abs(pred - tgt)
    bb = _B_L1
    small = 0.5 / bb * (bb * diff + 1.0) * jnp.log(bb * diff + 1.0) - 0.5 * diff
    big = 1.5 * diff + 1.5 / bb - 0.5
    l1 = jnp.where(diff < 1.0, small, big)
    sl_ref[...] = sl_ref[...] + jnp.sum(jnp.where(pos_row, l1, 0.0))


def _corr_kernel(xsel_ref, ov_ref, out_ref):
    x = xsel_ref[...]
    o = ov_ref[...]
    u = jnp.exp(-x)
    d = 1.0 + u
    lg = jnp.log2(d)
    el = lg * _LN2
    r2 = jnp.exp2(lg * -2.0)
    f_neg = (x + el) * r2
    f_pos = (u * u) * (r2 * el)
    act = o >= 0.4
    pos = o >= 0.5
    corr = jnp.where(act, jnp.where(pos, f_pos, 0.0) - f_neg, 0.0)
    s = jnp.sum(corr, axis=1, keepdims=True)
    out_ref[...] = jnp.sum(s, axis=0, keepdims=True)


def kernel(loc, conf, priors, targets):
    B, P, C = conf.shape
    NOBJ = targets.shape[1]
    PBLK = 4096
    NB = P // PBLK

    priors_t = priors.T
    loc_t = loc.transpose(0, 2, 1)
    tdat = jnp.pad(targets.transpose(0, 2, 1), ((0, 0), (0, 3), (0, 0)))

    ov, clsa, enc = pl.pallas_call(
        functools.partial(_match_kernel, nobj=NOBJ, num_p=P, num_c=C),
        grid=(B,),
        in_specs=[
            pl.BlockSpec((1, NOBJ, 5), lambda b: (b, 0, 0)),
            pl.BlockSpec((1, 8, NOBJ), lambda b: (b, 0, 0)),
            pl.BlockSpec((4, P), lambda b: (0, 0)),
        ],
        out_specs=[
            pl.BlockSpec((1, 1, P), lambda b: (b, 0, 0)),
            pl.BlockSpec((1, 1, P), lambda b: (b, 0, 0)),
            pl.BlockSpec((1, 4, P), lambda b: (b, 0, 0)),
        ],
        out_shape=[
            jax.ShapeDtypeStruct((B, 1, P), jnp.float32),
            jax.ShapeDtypeStruct((B, 1, P), jnp.int32),
            jax.ShapeDtypeStruct((B, 4, P), jnp.float32),
        ],
    )(targets, tdat, priors_t)

    clsa_col = clsa.reshape(B, P, 1)
    conf_flat = conf.reshape(B, NB, 1, PBLK * C)
    sl, sc, sp, xsel = pl.pallas_call(
        functools.partial(_loss_kernel, pblk=PBLK, num_c=C),
        grid=(B, NB),
        in_specs=[
            pl.BlockSpec((1, 1, 1, PBLK * C), lambda b, j: (b, j, 0, 0)),
            pl.BlockSpec((1, PBLK, C), lambda b, j: (b, j, 0)),
            pl.BlockSpec((1, 4, PBLK), lambda b, j: (b, 0, j)),
            pl.BlockSpec((1, 4, PBLK), lambda b, j: (b, 0, j)),
            pl.BlockSpec((1, 1, PBLK), lambda b, j: (b, 0, j)),
            pl.BlockSpec((1, PBLK, 1), lambda b, j: (b, j, 0)),
        ],
        out_specs=[
            pl.BlockSpec((1, 1), lambda b, j: (0, 0)),
            pl.BlockSpec((1, 1), lambda b, j: (0, 0)),
            pl.BlockSpec((1, 1), lambda b, j: (0, 0)),
            pl.BlockSpec((1, PBLK, 1), lambda b, j: (b, j, 0)),
        ],
        out_shape=[
            jax.ShapeDtypeStruct((1, 1), jnp.float32),
            jax.ShapeDtypeStruct((1, 1), jnp.float32),
            jax.ShapeDtypeStruct((1, 1), jnp.float32),
            jax.ShapeDtypeStruct((B, P, 1), jnp.float32),
        ],
    )(conf_flat, conf, loc_t, enc, ov, clsa_col)

    corr = pl.pallas_call(
        _corr_kernel,
        out_shape=jax.ShapeDtypeStruct((1, 1), jnp.float32),
    )(xsel.reshape(B, P), ov.reshape(B, P))

    npos = jnp.maximum(sp[0, 0], 1.0)
    loss_l = sl[0, 0] / npos
    loss_c = (sc[0, 0] + corr[0, 0]) / npos
    ov2 = ov.reshape(B, P)
    pos = ov2 >= 0.5
    neg = ov2 < 0.4
    return loss_l, loss_c, pos, neg


# speedup vs baseline: 1.8446x; 1.8446x over previous
import functools

import jax
import jax.numpy as jnp
import numpy as np
from jax.experimental import pallas as pl
from jax.experimental.pallas import tpu as pltpu


def _match_kernel(targets_ref, tdat_ref, priors_ref, loc_ref, ov_ref,
                  clsa_ref, posf_ref, enc_ref, loct_ref, *, nobj, num_p):
    t = targets_ref[0]
    pr = priors_ref[...]

    loct_ref[0] = jax.lax.dot_general(
        jnp.eye(4, dtype=jnp.float32), loc_ref[0],
        dimension_numbers=(((1,), (1,)), ((), ())),
        preferred_element_type=jnp.float32)

    tx1 = t[:, 0:1]
    ty1 = t[:, 1:2]
    tx2 = t[:, 2:3]
    ty2 = t[:, 3:4]

    pcx = pr[0:1, :]
    pcy = pr[1:2, :]
    pw = pr[2:3, :]
    ph = pr[3:4, :]
    px1 = pcx - pw * 0.5
    py1 = pcy - ph * 0.5
    px2 = pcx + pw * 0.5
    py2 = pcy + ph * 0.5

    iw = jnp.maximum(jnp.minimum(tx2, px2) - jnp.maximum(tx1, px1), 0.0)
    ih = jnp.maximum(jnp.minimum(ty2, py2) - jnp.maximum(ty1, py1), 0.0)
    inter = iw * ih
    area_t = (tx2 - tx1) * (ty2 - ty1)
    area_p = (px2 - px1) * (py2 - py1)
    union = jnp.maximum(area_t + area_p - inter, 1e-8)
    ov = inter / union

    iota_t = jax.lax.broadcasted_iota(jnp.int32, (nobj, num_p), 0)
    iota_p = jax.lax.broadcasted_iota(jnp.int32, (nobj, num_p), 1)

    btv = jnp.max(ov, axis=0, keepdims=True)
    bpv = jnp.max(ov, axis=1, keepdims=True)
    bpi = jnp.min(jnp.where(ov == bpv, iota_p, num_p), axis=1,
                  keepdims=True)

    oh2 = jnp.where(iota_p == bpi, 1.0, 0.0)
    rr = jax.lax.broadcasted_iota(jnp.int32, (2, nobj), 0)
    cc = jax.lax.broadcasted_iota(jnp.int32, (2, nobj), 1)
    e = cc - 16 + 16 * rr
    valid = jnp.logical_and(e >= 0, e < 16)
    pow2e = jax.lax.bitcast_convert_type(
        jax.lax.shift_left(e + 127, 23), jnp.float32)
    wmat = jnp.where(valid, pow2e, 0.0)
    s2 = jnp.dot(wmat, oh2, preferred_element_type=jnp.float32)
    s_hi = s2[0:1, :]
    s_lo = s2[1:2, :]
    forced = (s_hi + s_lo) > 0.0
    e_hi = jax.lax.shift_right_logical(
        jax.lax.bitcast_convert_type(s_hi, jnp.int32), 23) - (127 - 16)
    e_lo = jax.lax.shift_right_logical(
        jax.lax.bitcast_convert_type(s_lo, jnp.int32), 23) - 127
    tf_i = jnp.where(s_hi > 0.0, e_hi, e_lo)
    tf_i = jnp.where(forced, tf_i, -1)
    fov = jnp.where(forced, 2.0, btv)

    f_b = jnp.where(forced, 1.0, 0.0)
    mi_f = jnp.where(iota_t == tf_i, 1.0, 0.0)
    mv_f = jnp.where(ov == btv, 1.0, 0.0)
    oh = mv_f + f_b * (mi_f - mv_f)
    gathered = jnp.dot(tdat_ref[0], oh,
                       preferred_element_type=jnp.float32)
    g_x1 = gathered[0:1, :]
    g_y1 = gathered[1:2, :]
    g_x2 = gathered[2:3, :]
    g_y2 = gathered[3:4, :]
    g_lab = gathered[4:5, :]

    inv_pw = 1.0 / pw
    inv_ph = 1.0 / ph
    e_cx = ((g_x1 + g_x2) * 0.5 - pcx) * (10.0 * inv_pw)
    e_cy = ((g_y1 + g_y2) * 0.5 - pcy) * (10.0 * inv_ph)
    e_w = jnp.log(jnp.maximum((g_x2 - g_x1) * inv_pw, 1e-8)) * 5.0
    e_h = jnp.log(jnp.maximum((g_y2 - g_y1) * inv_ph, 1e-8)) * 5.0

    cls_i = (g_lab + 0.5).astype(jnp.int32)
    act = fov >= 0.4
    posr = fov >= 0.5
    neg1 = jnp.full((1, num_p), -1, jnp.int32)

    ov_ref[0] = fov
    clsa_ref[0] = jnp.where(act, cls_i, neg1)
    posf_ref[0] = jnp.where(posr, 1.0, 0.0)
    enc_ref[0, 0:1, :] = e_cx
    enc_ref[0, 1:2, :] = e_cy
    enc_ref[0, 2:3, :] = e_w
    enc_ref[0, 3:4, :] = e_h


_B_L1 = float(np.e ** (1.5 / 0.5) - 1.0)


_LN2 = float(np.log(2.0))


def _focal_contrib(x, clsa_col, posf_col, *, pblk, num_c):
    u = jnp.exp(-x)
    d = 1.0 + u
    lg = jnp.log2(d)
    el = lg * _LN2
    r2 = jnp.exp2(lg * -2.0)
    f_neg = (x + el) * r2
    f_pos = (u * u) * (r2 * el)

    iota_c = jax.lax.broadcasted_iota(jnp.int32, (pblk, num_c), 1)
    m_act = iota_c == clsa_col
    contrib = jnp.where(m_act, posf_col * f_pos, f_neg)
    return jnp.sum(contrib)


def _loss_kernel(conf0_ref, conf1_ref, loc_ref, enc_ref, ovr_ref, clsa_ref,
                 posf_ref, sl_ref, sc_ref, sp_ref, *, pblk, num_c):
    b = pl.program_id(0)
    j = pl.program_id(1)

    @pl.when(jnp.logical_and(b == 0, j == 0))
    def _init():
        zero11 = jnp.zeros((1, 1), jnp.float32)
        sl_ref[...] = zero11
        sc_ref[...] = zero11
        sp_ref[...] = zero11

    s0 = _focal_contrib(conf0_ref[0, 0, 0], clsa_ref[0, 0:pblk],
                        posf_ref[0, 0:pblk], pblk=pblk, num_c=num_c)
    s1 = _focal_contrib(conf1_ref[0, 0, 0], clsa_ref[0, pblk:],
                        posf_ref[0, pblk:], pblk=pblk, num_c=num_c)
    sc_ref[...] = sc_ref[...] + (s0 + s1)

    ov_row = ovr_ref[0]
    pos_row = ov_row >= 0.5
    sp_ref[...] = sp_ref[...] + jnp.sum(jnp.where(pos_row, 1.0, 0.0))

    pred = loc_ref[0]
    tgt = enc_ref[0]
    diff = jnp.abs(pred - tgt)
    bb = _B_L1
    small = 0.5 / bb * (bb * diff + 1.0) * jnp.log(bb * diff + 1.0) - 0.5 * diff
    big = 1.5 * diff + 1.5 / bb - 0.5
    l1 = jnp.where(diff < 1.0, small, big)
    sl_ref[...] = sl_ref[...] + jnp.sum(jnp.where(pos_row, l1, 0.0))


def kernel(loc, conf, priors, targets):
    B, P, C = conf.shape
    NOBJ = targets.shape[1]
    PBLK = 4096
    NB2 = P // (2 * PBLK)

    priors_t = priors.T
    tdat = jnp.pad(targets.transpose(0, 2, 1), ((0, 0), (0, 3), (0, 0)))

    ov, clsa, posf, enc, loct = pl.pallas_call(
        functools.partial(_match_kernel, nobj=NOBJ, num_p=P),
        grid=(B,),
        in_specs=[
            pl.BlockSpec((1, NOBJ, 5), lambda b: (b, 0, 0)),
            pl.BlockSpec((1, 8, NOBJ), lambda b: (b, 0, 0)),
            pl.BlockSpec((4, P), lambda b: (0, 0)),
            pl.BlockSpec((1, P, 4), lambda b: (b, 0, 0)),
        ],
        out_specs=[
            pl.BlockSpec((1, 1, P), lambda b: (b, 0, 0)),
            pl.BlockSpec((1, 1, P), lambda b: (b, 0, 0)),
            pl.BlockSpec((1, 1, P), lambda b: (b, 0, 0)),
            pl.BlockSpec((1, 4, P), lambda b: (b, 0, 0)),
            pl.BlockSpec((1, 4, P), lambda b: (b, 0, 0)),
        ],
        out_shape=[
            jax.ShapeDtypeStruct((B, 1, P), jnp.float32),
            jax.ShapeDtypeStruct((B, 1, P), jnp.int32),
            jax.ShapeDtypeStruct((B, 1, P), jnp.float32),
            jax.ShapeDtypeStruct((B, 4, P), jnp.float32),
            jax.ShapeDtypeStruct((B, 4, P), jnp.float32),
        ],
    )(targets, tdat, priors_t, loc)

    clsa_col = clsa.reshape(B, P, 1)
    posf_col = posf.reshape(B, P, 1)
    conf5 = conf.reshape(B, NB2, 2, PBLK, C)

    sl, sc, sp = pl.pallas_call(
        functools.partial(_loss_kernel, pblk=PBLK, num_c=C),
        grid=(B, NB2),
        in_specs=[
            pl.BlockSpec((1, 1, 1, PBLK, C), lambda b, j: (b, j, 0, 0, 0)),
            pl.BlockSpec((1, 1, 1, PBLK, C), lambda b, j: (b, j, 1, 0, 0)),
            pl.BlockSpec((1, 4, 2 * PBLK), lambda b, j: (b, 0, j)),
            pl.BlockSpec((1, 4, 2 * PBLK), lambda b, j: (b, 0, j)),
            pl.BlockSpec((1, 1, 2 * PBLK), lambda b, j: (b, 0, j)),
            pl.BlockSpec((1, 2 * PBLK, 1), lambda b, j: (b, j, 0)),
            pl.BlockSpec((1, 2 * PBLK, 1), lambda b, j: (b, j, 0)),
        ],
        out_specs=[
            pl.BlockSpec((1, 1), lambda b, j: (0, 0)),
            pl.BlockSpec((1, 1), lambda b, j: (0, 0)),
            pl.BlockSpec((1, 1), lambda b, j: (0, 0)),
        ],
        out_shape=[
            jax.ShapeDtypeStruct((1, 1), jnp.float32),
            jax.ShapeDtypeStruct((1, 1), jnp.float32),
            jax.ShapeDtypeStruct((1, 1), jnp.float32),
        ],
    )(conf5, conf5, loct, enc, ov, clsa_col, posf_col)

    npos = jnp.maximum(sp[0, 0], 1.0)
    loss_l = sl[0, 0] / npos
    loss_c = sc[0, 0] / npos
    ov2 = ov.reshape(B, P)
    pos = ov2 >= 0.5
    neg = ov2 < 0.4
    return loss_l, loss_c, pos, neg


# speedup vs baseline: 1.9838x; 1.0755x over previous
import functools

import jax
import jax.numpy as jnp
import numpy as np
from jax.experimental import pallas as pl
from jax.experimental.pallas import tpu as pltpu


def _match_kernel(targets_ref, tdat_ref, priors_ref, ov_ref, clsa_ref,
                  posf_ref, enc_ref, *, nobj, num_p):
    t = targets_ref[0]
    pr = priors_ref[...]

    tx1 = t[:, 0:1]
    ty1 = t[:, 1:2]
    tx2 = t[:, 2:3]
    ty2 = t[:, 3:4]

    pcx = pr[0:1, :]
    pcy = pr[1:2, :]
    pw = pr[2:3, :]
    ph = pr[3:4, :]
    px1 = pcx - pw * 0.5
    py1 = pcy - ph * 0.5
    px2 = pcx + pw * 0.5
    py2 = pcy + ph * 0.5

    iw = jnp.maximum(jnp.minimum(tx2, px2) - jnp.maximum(tx1, px1), 0.0)
    ih = jnp.maximum(jnp.minimum(ty2, py2) - jnp.maximum(ty1, py1), 0.0)
    inter = iw * ih
    area_t = (tx2 - tx1) * (ty2 - ty1)
    area_p = (px2 - px1) * (py2 - py1)
    union = jnp.maximum(area_t + area_p - inter, 1e-8)
    ov = inter / union

    iota_t = jax.lax.broadcasted_iota(jnp.int32, (nobj, num_p), 0)
    iota_p = jax.lax.broadcasted_iota(jnp.int32, (nobj, num_p), 1)

    btv = jnp.max(ov, axis=0, keepdims=True)
    bpv = jnp.max(ov, axis=1, keepdims=True)
    bpi = jnp.min(jnp.where(ov == bpv, iota_p, num_p), axis=1,
                  keepdims=True)

    oh2 = jnp.where(iota_p == bpi, 1.0, 0.0)
    rr = jax.lax.broadcasted_iota(jnp.int32, (2, nobj), 0)
    cc = jax.lax.broadcasted_iota(jnp.int32, (2, nobj), 1)
    e = cc - 16 + 16 * rr
    valid = jnp.logical_and(e >= 0, e < 16)
    pow2e = jax.lax.bitcast_convert_type(
        jax.lax.shift_left(e + 127, 23), jnp.float32)
    wmat = jnp.where(valid, pow2e, 0.0)
    s2 = jnp.dot(wmat, oh2, preferred_element_type=jnp.float32)
    s_hi = s2[0:1, :]
    s_lo = s2[1:2, :]
    forced = (s_hi + s_lo) > 0.0
    e_hi = jax.lax.shift_right_logical(
        jax.lax.bitcast_convert_type(s_hi, jnp.int32), 23) - (127 - 16)
    e_lo = jax.lax.shift_right_logical(
        jax.lax.bitcast_convert_type(s_lo, jnp.int32), 23) - 127
    tf_i = jnp.where(s_hi > 0.0, e_hi, e_lo)
    tf_i = jnp.where(forced, tf_i, -1)
    fov = jnp.where(forced, 2.0, btv)

    f_b = jnp.where(forced, 1.0, 0.0)
    mi_f = jnp.where(iota_t == tf_i, 1.0, 0.0)
    mv_f = jnp.where(ov == btv, 1.0, 0.0)
    oh = mv_f + f_b * (mi_f - mv_f)
    gathered = jnp.dot(tdat_ref[0], oh,
                       preferred_element_type=jnp.float32)
    g_x1 = gathered[0:1, :]
    g_y1 = gathered[1:2, :]
    g_x2 = gathered[2:3, :]
    g_y2 = gathered[3:4, :]
    g_lab = gathered[4:5, :]

    inv_pw = 1.0 / pw
    inv_ph = 1.0 / ph
    e_cx = ((g_x1 + g_x2) * 0.5 - pcx) * (10.0 * inv_pw)
    e_cy = ((g_y1 + g_y2) * 0.5 - pcy) * (10.0 * inv_ph)
    e_w = jnp.log(jnp.maximum((g_x2 - g_x1) * inv_pw, 1e-8)) * 5.0
    e_h = jnp.log(jnp.maximum((g_y2 - g_y1) * inv_ph, 1e-8)) * 5.0

    cls_i = (g_lab + 0.5).astype(jnp.int32)
    act = fov >= 0.4
    posr = fov >= 0.5
    neg1 = jnp.full((1, num_p), -1, jnp.int32)

    ov_ref[0] = fov
    clsa_ref[0] = jnp.where(act, cls_i, neg1)
    posf_ref[0] = jnp.where(posr, 1.0, 0.0)
    enc_ref[0, 0:1, :] = e_cx
    enc_ref[0, 1:2, :] = e_cy
    enc_ref[0, 2:3, :] = e_w
    enc_ref[0, 3:4, :] = e_h


_B_L1 = float(np.e ** (1.5 / 0.5) - 1.0)


_LN2 = float(np.log(2.0))


def _loss_kernel(conf_ref, loc_ref, enc_ref, ovr_ref, clsa_ref, posf_ref,
                 sl_ref, sc_ref, sp_ref, *, pblk, num_c):
    b = pl.program_id(0)
    j = pl.program_id(1)

    @pl.when(jnp.logical_and(b == 0, j == 0))
    def _init():
        zero11 = jnp.zeros((1, 1), jnp.float32)
        sl_ref[...] = zero11
        sc_ref[...] = zero11
        sp_ref[...] = zero11

    x = conf_ref[0]

    u = jnp.exp(-x)
    d = 1.0 + u
    lg = jnp.log2(d)
    el = lg * _LN2
    r2 = jnp.exp2(lg * -2.0)
    f_neg = (x + el) * r2
    f_pos = (u * u) * (r2 * el)

    iota_c = jax.lax.broadcasted_iota(jnp.int32, (pblk, num_c), 1)
    m_act = iota_c == clsa_ref[0]
    contrib = jnp.where(m_act, posf_ref[0] * f_pos, f_neg)
    sc_ref[...] = sc_ref[...] + jnp.sum(contrib)

    ov_row = ovr_ref[0]
    pos_row = ov_row >= 0.5
    sp_ref[...] = sp_ref[...] + jnp.sum(jnp.where(pos_row, 1.0, 0.0))

    pred = loc_ref[0]
    tgt = enc_ref[0]
    diff = jnp.abs(pred - tgt)
    bb = _B_L1
    small = 0.5 / bb * (bb * diff + 1.0) * jnp.log(bb * diff + 1.0) - 0.5 * diff
    big = 1.5 * diff + 1.5 / bb - 0.5
    l1 = jnp.where(diff < 1.0, small, big)
    sl_ref[...] = sl_ref[...] + jnp.sum(jnp.where(pos_row, l1, 0.0))


def kernel(loc, conf, priors, targets):
    B, P, C = conf.shape
    NOBJ = targets.shape[1]
    PBLK = 8192
    NB = P // PBLK

    priors_t = priors.T
    loc_t = loc.transpose(0, 2, 1)
    tdat = jnp.pad(targets.transpose(0, 2, 1), ((0, 0), (0, 3), (0, 0)))

    ov, clsa, posf, enc = pl.pallas_call(
        functools.partial(_match_kernel, nobj=NOBJ, num_p=P),
        grid=(B,),
        in_specs=[
            pl.BlockSpec((1, NOBJ, 5), lambda b: (b, 0, 0)),
            pl.BlockSpec((1, 8, NOBJ), lambda b: (b, 0, 0)),
            pl.BlockSpec((4, P), lambda b: (0, 0)),
        ],
        out_specs=[
            pl.BlockSpec((1, 1, P), lambda b: (b, 0, 0)),
            pl.BlockSpec((1, 1, P), lambda b: (b, 0, 0)),
            pl.BlockSpec((1, 1, P), lambda b: (b, 0, 0)),
            pl.BlockSpec((1, 4, P), lambda b: (b, 0, 0)),
        ],
        out_shape=[
            jax.ShapeDtypeStruct((B, 1, P), jnp.float32),
            jax.ShapeDtypeStruct((B, 1, P), jnp.int32),
            jax.ShapeDtypeStruct((B, 1, P), jnp.float32),
            jax.ShapeDtypeStruct((B, 4, P), jnp.float32),
        ],
    )(targets, tdat, priors_t)

    clsa_col = clsa.reshape(B, P, 1)
    posf_col = posf.reshape(B, P, 1)

    sl, sc, sp = pl.pallas_call(
        functools.partial(_loss_kernel, pblk=PBLK, num_c=C),
        grid=(B, NB),
        in_specs=[
            pl.BlockSpec((1, PBLK, C), lambda b, j: (b, j, 0)),
            pl.BlockSpec((1, 4, PBLK), lambda b, j: (b, 0, j)),
            pl.BlockSpec((1, 4, PBLK), lambda b, j: (b, 0, j)),
            pl.BlockSpec((1, 1, PBLK), lambda b, j: (b, 0, j)),
            pl.BlockSpec((1, PBLK, 1), lambda b, j: (b, j, 0)),
            pl.BlockSpec((1, PBLK, 1), lambda b, j: (b, j, 0)),
        ],
        out_specs=[
            pl.BlockSpec((1, 1), lambda b, j: (0, 0)),
            pl.BlockSpec((1, 1), lambda b, j: (0, 0)),
            pl.BlockSpec((1, 1), lambda b, j: (0, 0)),
        ],
        out_shape=[
            jax.ShapeDtypeStruct((1, 1), jnp.float32),
            jax.ShapeDtypeStruct((1, 1), jnp.float32),
            jax.ShapeDtypeStruct((1, 1), jnp.float32),
        ],
    )(conf, loc_t, enc, ov, clsa_col, posf_col)

    npos = jnp.maximum(sp[0, 0], 1.0)
    loss_l = sl[0, 0] / npos
    loss_c = sc[0, 0] / npos
    ov2 = ov.reshape(B, P)
    pos = ov2 >= 0.5
    neg = ov2 < 0.4
    return loss_l, loss_c, pos, neg


# speedup vs baseline: 2.0384x; 1.0275x over previous
import functools

import jax
import jax.numpy as jnp
import numpy as np
from jax.experimental import pallas as pl
from jax.experimental.pallas import tpu as pltpu


def _match_kernel(targets_ref, tdat_ref, priors_ref, ov_ref, clsa_ref,
                  posf_ref, enc_ref, *, nobj, num_p):
    t = targets_ref[0]
    pr = priors_ref[...]

    tx1 = t[:, 0:1]
    ty1 = t[:, 1:2]
    tx2 = t[:, 2:3]
    ty2 = t[:, 3:4]

    pcx = pr[0:1, :]
    pcy = pr[1:2, :]
    pw = pr[2:3, :]
    ph = pr[3:4, :]
    px1 = pcx - pw * 0.5
    py1 = pcy - ph * 0.5
    px2 = pcx + pw * 0.5
    py2 = pcy + ph * 0.5

    iw = jnp.maximum(jnp.minimum(tx2, px2) - jnp.maximum(tx1, px1), 0.0)
    ih = jnp.maximum(jnp.minimum(ty2, py2) - jnp.maximum(ty1, py1), 0.0)
    inter = iw * ih
    area_t = (tx2 - tx1) * (ty2 - ty1)
    area_p = (px2 - px1) * (py2 - py1)
    union = jnp.maximum(area_t + area_p - inter, 1e-8)
    ov = inter / union

    iota_t = jax.lax.broadcasted_iota(jnp.int32, (nobj, num_p), 0)
    iota_p = jax.lax.broadcasted_iota(jnp.int32, (nobj, num_p), 1)

    btv = jnp.max(ov, axis=0, keepdims=True)
    bpv = jnp.max(ov, axis=1, keepdims=True)
    bpi = jnp.min(jnp.where(ov == bpv, iota_p, num_p), axis=1,
                  keepdims=True)

    oh2 = jnp.where(iota_p == bpi, 1.0, 0.0)
    rr = jax.lax.broadcasted_iota(jnp.int32, (2, nobj), 0)
    cc = jax.lax.broadcasted_iota(jnp.int32, (2, nobj), 1)
    e = cc - 16 + 16 * rr
    valid = jnp.logical_and(e >= 0, e < 16)
    pow2e = jax.lax.bitcast_convert_type(
        jax.lax.shift_left(e + 127, 23), jnp.float32)
    wmat = jnp.where(valid, pow2e, 0.0)
    s2 = jnp.dot(wmat, oh2, preferred_element_type=jnp.float32)
    s_hi = s2[0:1, :]
    s_lo = s2[1:2, :]
    forced = (s_hi + s_lo) > 0.0
    e_hi = jax.lax.shift_right_logical(
        jax.lax.bitcast_convert_type(s_hi, jnp.int32), 23) - (127 - 16)
    e_lo = jax.lax.shift_right_logical(
        jax.lax.bitcast_convert_type(s_lo, jnp.int32), 23) - 127
    tf_i = jnp.where(s_hi > 0.0, e_hi, e_lo)
    tf_i = jnp.where(forced, tf_i, -1)
    fov = jnp.where(forced, 2.0, btv)

    f_b = jnp.where(forced, 1.0, 0.0)
    mi_f = jnp.where(iota_t == tf_i, 1.0, 0.0)
    mv_f = jnp.where(ov == btv, 1.0, 0.0)
    oh = mv_f + f_b * (mi_f - mv_f)
    gathered = jnp.dot(tdat_ref[0], oh,
                       preferred_element_type=jnp.float32)
    g_x1 = gathered[0:1, :]
    g_y1 = gathered[1:2, :]
    g_x2 = gathered[2:3, :]
    g_y2 = gathered[3:4, :]
    g_lab = gathered[4:5, :]

    inv_pw = 1.0 / pw
    inv_ph = 1.0 / ph
    e_cx = ((g_x1 + g_x2) * 0.5 - pcx) * (10.0 * inv_pw)
    e_cy = ((g_y1 + g_y2) * 0.5 - pcy) * (10.0 * inv_ph)
    e_w = jnp.log(jnp.maximum((g_x2 - g_x1) * inv_pw, 1e-8)) * 5.0
    e_h = jnp.log(jnp.maximum((g_y2 - g_y1) * inv_ph, 1e-8)) * 5.0

    cls_i = (g_lab + 0.5).astype(jnp.int32)
    act = fov >= 0.4
    posr = fov >= 0.5
    neg1 = jnp.full((1, num_p), -1, jnp.int32)

    ov_ref[0] = fov
    clsa_ref[0] = jnp.where(act, cls_i, neg1)
    posf_ref[0] = jnp.where(posr, 1.0, 0.0)
    enc_ref[0, 0:1, :] = e_cx
    enc_ref[0, 1:2, :] = e_cy
    enc_ref[0, 2:3, :] = e_w
    enc_ref[0, 3:4, :] = e_h


_B_L1 = float(np.e ** (1.5 / 0.5) - 1.0)


_LN2 = float(np.log(2.0))


def _loss_kernel(conf_ref, loc_ref, enc_ref, ovr_ref, clsa_ref, posf_ref,
                 sl_ref, sc_ref, sp_ref, *, pblk, num_c):
    b = pl.program_id(0)
    j = pl.program_id(1)

    @pl.when(jnp.logical_and(b == 0, j == 0))
    def _init():
        zero11 = jnp.zeros((1, 1), jnp.float32)
        sl_ref[...] = zero11
        sc_ref[...] = zero11
        sp_ref[...] = zero11

    x = conf_ref[0]

    u = jnp.exp(-x)
    d = 1.0 + u
    lg = jnp.log2(d)
    el = lg * _LN2
    r2 = jnp.exp2(lg * -2.0)
    f_neg = (x + el) * r2
    f_pos = (u * u) * (r2 * el)

    iota_c = jax.lax.broadcasted_iota(jnp.int32, (pblk, num_c), 1)
    m_act = iota_c == clsa_ref[0]
    contrib = jnp.where(m_act, posf_ref[0] * f_pos, f_neg)
    sc_ref[...] = sc_ref[...] + jnp.sum(contrib)

    ov_row = ovr_ref[0]
    pos_row = ov_row >= 0.5
    sp_ref[...] = sp_ref[...] + jnp.sum(jnp.where(pos_row, 1.0, 0.0))

    pred = loc_ref[0]
    tgt = enc_ref[0]
    diff = jnp.abs(pred - tgt)
    bb = _B_L1
    small = 0.5 / bb * (bb * diff + 1.0) * jnp.log(bb * diff + 1.0) - 0.5 * diff
    big = 1.5 * diff + 1.5 / bb - 0.5
    l1 = jnp.where(diff < 1.0, small, big)
    sl_ref[...] = sl_ref[...] + jnp.sum(jnp.where(pos_row, l1, 0.0))


def kernel(loc, conf, priors, targets):
    B, P, C = conf.shape
    NOBJ = targets.shape[1]
    PBLK = 16384
    NB = P // PBLK

    priors_t = priors.T
    loc_t = loc.transpose(0, 2, 1)
    tdat = jnp.pad(targets.transpose(0, 2, 1), ((0, 0), (0, 3), (0, 0)))

    ov, clsa, posf, enc = pl.pallas_call(
        functools.partial(_match_kernel, nobj=NOBJ, num_p=P),
        grid=(B,),
        in_specs=[
            pl.BlockSpec((1, NOBJ, 5), lambda b: (b, 0, 0)),
            pl.BlockSpec((1, 8, NOBJ), lambda b: (b, 0, 0)),
            pl.BlockSpec((4, P), lambda b: (0, 0)),
        ],
        out_specs=[
            pl.BlockSpec((1, 1, P), lambda b: (b, 0, 0)),
            pl.BlockSpec((1, 1, P), lambda b: (b, 0, 0)),
            pl.BlockSpec((1, 1, P), lambda b: (b, 0, 0)),
            pl.BlockSpec((1, 4, P), lambda b: (b, 0, 0)),
        ],
        out_shape=[
            jax.ShapeDtypeStruct((B, 1, P), jnp.float32),
            jax.ShapeDtypeStruct((B, 1, P), jnp.int32),
            jax.ShapeDtypeStruct((B, 1, P), jnp.float32),
            jax.ShapeDtypeStruct((B, 4, P), jnp.float32),
        ],
    )(targets, tdat, priors_t)

    clsa_col = clsa.reshape(B, P, 1)
    posf_col = posf.reshape(B, P, 1)

    sl, sc, sp = pl.pallas_call(
        functools.partial(_loss_kernel, pblk=PBLK, num_c=C),
        grid=(B, NB),
        in_specs=[
            pl.BlockSpec((1, PBLK, C), lambda b, j: (b, j, 0)),
            pl.BlockSpec((1, 4, PBLK), lambda b, j: (b, 0, j)),
            pl.BlockSpec((1, 4, PBLK), lambda b, j: (b, 0, j)),
            pl.BlockSpec((1, 1, PBLK), lambda b, j: (b, 0, j)),
            pl.BlockSpec((1, PBLK, 1), lambda b, j: (b, j, 0)),
            pl.BlockSpec((1, PBLK, 1), lambda b, j: (b, j, 0)),
        ],
        out_specs=[
            pl.BlockSpec((1, 1), lambda b, j: (0, 0)),
            pl.BlockSpec((1, 1), lambda b, j: (0, 0)),
            pl.BlockSpec((1, 1), lambda b, j: (0, 0)),
        ],
        out_shape=[
            jax.ShapeDtypeStruct((1, 1), jnp.float32),
            jax.ShapeDtypeStruct((1, 1), jnp.float32),
            jax.ShapeDtypeStruct((1, 1), jnp.float32),
        ],
    )(conf, loc_t, enc, ov, clsa_col, posf_col)

    npos = jnp.maximum(sp[0, 0], 1.0)
    loss_l = sl[0, 0] / npos
    loss_c = sc[0, 0] / npos
    ov2 = ov.reshape(B, P)
    pos = ov2 >= 0.5
    neg = ov2 < 0.4
    return loss_l, loss_c, pos, neg


# speedup vs baseline: 2.0449x; 1.0032x over previous
import functools

import jax
import jax.numpy as jnp
import numpy as np
from jax.experimental import pallas as pl


def _match_kernel(targets_ref, tdat_ref, priors_ref, ov_ref, clsa_ref,
                  posf_ref, enc_ref, *, nobj, num_p):
    t = targets_ref[0]
    pr = priors_ref[...]

    tx1 = t[:, 0:1]
    ty1 = t[:, 1:2]
    tx2 = t[:, 2:3]
    ty2 = t[:, 3:4]

    pcx = pr[0:1, :]
    pcy = pr[1:2, :]
    pw = pr[2:3, :]
    ph = pr[3:4, :]
    px1 = pcx - pw * 0.5
    py1 = pcy - ph * 0.5
    px2 = pcx + pw * 0.5
    py2 = pcy + ph * 0.5

    iw = jnp.maximum(jnp.minimum(tx2, px2) - jnp.maximum(tx1, px1), 0.0)
    ih = jnp.maximum(jnp.minimum(ty2, py2) - jnp.maximum(ty1, py1), 0.0)
    inter = iw * ih
    area_t = (tx2 - tx1) * (ty2 - ty1)
    area_p = (px2 - px1) * (py2 - py1)
    union = jnp.maximum(area_t + area_p - inter, 1e-8)
    ov = inter / union

    iota_t = jax.lax.broadcasted_iota(jnp.int32, (nobj, num_p), 0)
    iota_p = jax.lax.broadcasted_iota(jnp.int32, (nobj, num_p), 1)

    btv = jnp.max(ov, axis=0, keepdims=True)
    bpv = jnp.max(ov, axis=1, keepdims=True)
    bpi = jnp.min(jnp.where(ov == bpv, iota_p, num_p), axis=1,
                  keepdims=True)

    oh2 = jnp.where(iota_p == bpi, 1.0, 0.0)
    rr = jax.lax.broadcasted_iota(jnp.int32, (2, nobj), 0)
    cc = jax.lax.broadcasted_iota(jnp.int32, (2, nobj), 1)
    e = cc - 16 + 16 * rr
    valid = jnp.logical_and(e >= 0, e < 16)
    pow2e = jax.lax.bitcast_convert_type(
        jax.lax.shift_left(e + 127, 23), jnp.float32)
    wmat = jnp.where(valid, pow2e, 0.0)
    s2 = jnp.dot(wmat, oh2, preferred_element_type=jnp.float32)
    s_hi = s2[0:1, :]
    s_lo = s2[1:2, :]
    forced = (s_hi + s_lo) > 0.0
    e_hi = jax.lax.shift_right_logical(
        jax.lax.bitcast_convert_type(s_hi, jnp.int32), 23) - (127 - 16)
    e_lo = jax.lax.shift_right_logical(
        jax.lax.bitcast_convert_type(s_lo, jnp.int32), 23) - 127
    tf_i = jnp.where(s_hi > 0.0, e_hi, e_lo)
    tf_i = jnp.where(forced, tf_i, -1)
    fov = jnp.where(forced, 2.0, btv)

    f_b = jnp.where(forced, 1.0, 0.0)
    mi_f = jnp.where(iota_t == tf_i, 1.0, 0.0)
    mv_f = jnp.where(ov == btv, 1.0, 0.0)
    oh = mv_f + f_b * (mi_f - mv_f)
    gathered = jnp.dot(tdat_ref[0], oh,
                       preferred_element_type=jnp.float32)
    g_x1 = gathered[0:1, :]
    g_y1 = gathered[1:2, :]
    g_x2 = gathered[2:3, :]
    g_y2 = gathered[3:4, :]
    g_lab = gathered[4:5, :]

    inv_pw = 1.0 / pw
    inv_ph = 1.0 / ph
    e_cx = ((g_x1 + g_x2) * 0.5 - pcx) * (10.0 * inv_pw)
    e_cy = ((g_y1 + g_y2) * 0.5 - pcy) * (10.0 * inv_ph)
    e_w = jnp.log(jnp.maximum((g_x2 - g_x1) * inv_pw, 1e-8)) * 5.0
    e_h = jnp.log(jnp.maximum((g_y2 - g_y1) * inv_ph, 1e-8)) * 5.0

    cls_i = (g_lab + 0.5).astype(jnp.int32)
    act = fov >= 0.4
    posr = fov >= 0.5
    neg1 = jnp.full((1, num_p), -1, jnp.int32)

    ov_ref[0] = fov
    clsa_ref[0] = jnp.where(act, cls_i, neg1)
    posf_ref[0] = jnp.where(posr, 1.0, 0.0)
    enc_ref[0, 0:1, :] = e_cx
    enc_ref[0, 1:2, :] = e_cy
    enc_ref[0, 2:3, :] = e_w
    enc_ref[0, 3:4, :] = e_h


_B_L1 = float(np.e ** (1.5 / 0.5) - 1.0)


_LN2 = float(np.log(2.0))


def _loss_kernel(conf_ref, loc_ref, enc_ref, ovr_ref, clsa_ref, posf_ref,
                 sl_ref, sc_ref, sp_ref, *, pblk, num_c):
    b = pl.program_id(0)
    j = pl.program_id(1)

    @pl.when(jnp.logical_and(b == 0, j == 0))
    def _init():
        zero11 = jnp.zeros((1, 1), jnp.float32)
        sl_ref[...] = zero11
        sc_ref[...] = zero11
        sp_ref[...] = zero11

    x = conf_ref[0]

    u = jnp.exp(-x)
    d = 1.0 + u
    lg = jnp.log2(d)
    el = lg * _LN2
    r2 = jnp.exp2(lg * -2.0)
    f_neg = (x + el) * r2
    f_pos = (u * u) * (r2 * el)

    iota_c = jax.lax.broadcasted_iota(jnp.int32, (pblk, num_c), 1)
    m_act = iota_c == clsa_ref[0]
    contrib = jnp.where(m_act, posf_ref[0] * f_pos, f_neg)
    sc_ref[...] = sc_ref[...] + jnp.sum(contrib)

    ov_row = ovr_ref[0]
    pos_row = ov_row >= 0.5
    sp_ref[...] = sp_ref[...] + jnp.sum(jnp.where(pos_row, 1.0, 0.0))

    pred = loc_ref[0]
    tgt = enc_ref[0]
    diff = jnp.abs(pred - tgt)
    bb = _B_L1
    small = 0.5 / bb * (bb * diff + 1.0) * jnp.log(bb * diff + 1.0) - 0.5 * diff
    big = 1.5 * diff + 1.5 / bb - 0.5
    l1 = jnp.where(diff < 1.0, small, big)
    sl_ref[...] = sl_ref[...] + jnp.sum(jnp.where(pos_row, l1, 0.0))


def kernel(loc, conf, priors, targets):
    B, P, C = conf.shape
    NOBJ = targets.shape[1]
    PBLK = 16384
    NB = P // PBLK

    priors_t = priors.T
    loc_t = loc.transpose(0, 2, 1)
    tdat = jnp.pad(targets.transpose(0, 2, 1), ((0, 0), (0, 3), (0, 0)))

    ov, clsa, posf, enc = pl.pallas_call(
        functools.partial(_match_kernel, nobj=NOBJ, num_p=P),
        grid=(B,),
        in_specs=[
            pl.BlockSpec((1, NOBJ, 5), lambda b: (b, 0, 0)),
            pl.BlockSpec((1, 8, NOBJ), lambda b: (b, 0, 0)),
            pl.BlockSpec((4, P), lambda b: (0, 0)),
        ],
        out_specs=[
            pl.BlockSpec((1, 1, P), lambda b: (b, 0, 0)),
            pl.BlockSpec((1, 1, P), lambda b: (b, 0, 0)),
            pl.BlockSpec((1, 1, P), lambda b: (b, 0, 0)),
            pl.BlockSpec((1, 4, P), lambda b: (b, 0, 0)),
        ],
        out_shape=[
            jax.ShapeDtypeStruct((B, 1, P), jnp.float32),
            jax.ShapeDtypeStruct((B, 1, P), jnp.int32),
            jax.ShapeDtypeStruct((B, 1, P), jnp.float32),
            jax.ShapeDtypeStruct((B, 4, P), jnp.float32),
        ],
    )(targets, tdat, priors_t)

    clsa_col = clsa.reshape(B, P, 1)
    posf_col = posf.reshape(B, P, 1)

    sl, sc, sp = pl.pallas_call(
        functools.partial(_loss_kernel, pblk=PBLK, num_c=C),
        grid=(B, NB),
        in_specs=[
            pl.BlockSpec((1, PBLK, C), lambda b, j: (b, j, 0)),
            pl.BlockSpec((1, 4, PBLK), lambda b, j: (b, 0, j)),
            pl.BlockSpec((1, 4, PBLK), lambda b, j: (b, 0, j)),
            pl.BlockSpec((1, 1, PBLK), lambda b, j: (b, 0, j)),
            pl.BlockSpec((1, PBLK, 1), lambda b, j: (b, j, 0)),
            pl.BlockSpec((1, PBLK, 1), lambda b, j: (b, j, 0)),
        ],
        out_specs=[
            pl.BlockSpec((1, 1), lambda b, j: (0, 0)),
            pl.BlockSpec((1, 1), lambda b, j: (0, 0)),
            pl.BlockSpec((1, 1), lambda b, j: (0, 0)),
        ],
        out_shape=[
            jax.ShapeDtypeStruct((1, 1), jnp.float32),
            jax.ShapeDtypeStruct((1, 1), jnp.float32),
            jax.ShapeDtypeStruct((1, 1), jnp.float32),
        ],
    )(conf, loc_t, enc, ov, clsa_col, posf_col)

    npos = jnp.maximum(sp[0, 0], 1.0)
    loss_l = sl[0, 0] / npos
    loss_c = sc[0, 0] / npos
    ov2 = ov.reshape(B, P)
    pos = ov2 >= 0.5
    neg = ov2 < 0.4
    return loss_l, loss_c, pos, neg
